# baseline (speedup 1.0000x reference)
import functools
import math

import jax
import jax.numpy as jnp
from jax import lax
from jax.experimental import pallas as pl
from jax.experimental.pallas import tpu as pltpu
from jax.experimental.pallas import tpu_sc as plsc

RADIUS_EARTH = 6371.0
MIN_DIST = 1.0
MAX_DIST = 1000.0
N_DIST = 1024
EMB_DIM = 16
LOG_MIN = math.log(MIN_DIST / RADIUS_EARTH)
LOG_MAX = math.log(MAX_DIST / RADIUS_EARTH)

_N = 2048
_B = _N * _N
_ROW_BLK = 128
_V = N_DIST + 1
_NPAIR = EMB_DIM // 2
_TPACK = _NPAIR * _V

_NC = 2
_NS = 16
_NW = _NC * _NS
_BPW = _B // _NW
_CHUNK = 2048
_GRP = _CHUNK // 16
_NCHUNK = _BPW // _CHUNK
_PAD = EMB_DIM + 1


def _bucket_idx(d):
    sgn = jnp.sign(d)
    t = jnp.log(jnp.abs(d))
    t = jnp.clip(t, LOG_MIN, LOG_MAX)
    t = (t - LOG_MIN) / (LOG_MAX - LOG_MIN)
    t = t * (N_DIST / 2.0)
    half = N_DIST / 2.0
    t = jnp.where(sgn < 0, t + (half - 1.0), half - 1.0 - t)
    idx = t.astype(jnp.int32)
    idx = jnp.where(idx < 0, idx + (N_DIST + 1), idx)
    return idx


def _idx_body(dlon_ref, dlat_ref, ilon_ref, ilat_ref):
    ilon_ref[...] = _bucket_idx(dlon_ref[...])
    ilat_ref[...] = _bucket_idx(dlat_ref[...])


def _compute_indices(d_mat_lon, d_mat_lat):
    spec = pl.BlockSpec((_ROW_BLK, _N), lambda i: (i, 0))
    return pl.pallas_call(
        _idx_body,
        grid=(_N // _ROW_BLK,),
        in_specs=[spec, spec],
        out_specs=[spec, spec],
        out_shape=[
            jax.ShapeDtypeStruct((_N, _N), jnp.int32),
            jax.ShapeDtypeStruct((_N, _N), jnp.int32),
        ],
    )(d_mat_lon, d_mat_lat)


def _sc_gather_sum(tlon_flat, tlat_flat, idx_lon, idx_lat):
    mesh = plsc.VectorSubcoreMesh(core_axis_name="c", subcore_axis_name="s")

    @functools.partial(
        pl.kernel,
        out_type=jax.ShapeDtypeStruct((_N, EMB_DIM, _N), jnp.float32),
        mesh=mesh,
        compiler_params=pltpu.CompilerParams(needs_layout_passes=False),
        scratch_types=[
            pltpu.VMEM((_TPACK,), jnp.int32),
            pltpu.VMEM((_TPACK,), jnp.int32),
            pltpu.VMEM((_CHUNK,), jnp.int32),
            pltpu.VMEM((_CHUNK,), jnp.int32),
            pltpu.VMEM((_CHUNK,), jnp.int32),
            pltpu.VMEM((_CHUNK,), jnp.int32),
            pltpu.VMEM((2, EMB_DIM, _CHUNK), jnp.float32),
            pltpu.SemaphoreType.DMA,
            pltpu.SemaphoreType.DMA,
            pltpu.SemaphoreType.DMA,
            pltpu.SemaphoreType.DMA,
        ],
    )
    def k(tlon_hbm, tlat_hbm, ilon_hbm, ilat_hbm, out_hbm,
          tlon_v, tlat_v, ilon0, ilon1, ilat0, ilat1, obuf,
          si0, si1, so0, so1):
        si = (si0, si1)
        so = (so0, so1)
        ilon = (ilon0, ilon1)
        ilat = (ilat0, ilat1)
        wid = lax.axis_index("s") * _NC + lax.axis_index("c")
        base = wid * _BPW

        pltpu.sync_copy(tlon_hbm, tlon_v)
        pltpu.sync_copy(tlat_hbm, tlat_v)

        def idx_src(c):
            off = pl.multiple_of(base + c * _CHUNK, _CHUNK)
            return (ilon_hbm.at[pl.ds(off, _CHUNK)],
                    ilat_hbm.at[pl.ds(off, _CHUNK)])

        def out_dst(c):
            return out_hbm.at[wid * _NCHUNK + c]

        for b in (0, 1):
            slon, slat = idx_src(b)
            pltpu.async_copy(slon, ilon[b], si[b])
            pltpu.async_copy(slat, ilat[b], si[b])

        @pl.loop(0, _NCHUNK, step=2)
        def _pair(c0):
            for b in (0, 1):
                c = c0 + b
                slon, slat = idx_src(c)
                pltpu.make_async_copy(slon, ilon[b], si[b]).wait()
                pltpu.make_async_copy(slat, ilat[b], si[b]).wait()

                @pl.when(c >= 2)
                def _drain():
                    pltpu.make_async_copy(obuf.at[b], out_dst(c - 2),
                                          so[b]).wait()

                @plsc.parallel_loop(0, _GRP, 1, unroll=8)
                def _grp(g):
                    ivl = ilon[b][pl.ds(g * 16, 16)]
                    ivt = ilat[b][pl.ds(g * 16, 16)]
                    for d2 in range(_NPAIR):
                        gl = plsc.load_gather(tlon_v, [ivl + d2 * _V])
                        gt = plsc.load_gather(tlat_v, [ivt + d2 * _V])
                        s = (plsc.bitcast(gl, jnp.bfloat16)
                             + plsc.bitcast(gt, jnp.bfloat16))
                        s0, s1 = plsc.unpack(
                            s, format=plsc.PackFormat.INTERLEAVED,
                            preferred_element_type=jnp.float32)
                        obuf[b, 2 * d2, pl.ds(g * 16, 16)] = s0
                        obuf[b, 2 * d2 + 1, pl.ds(g * 16, 16)] = s1

                pltpu.async_copy(obuf.at[b], out_dst(c), so[b])

                @pl.when(c + 2 < _NCHUNK)
                def _prefetch():
                    nlon, nlat = idx_src(c + 2)
                    pltpu.async_copy(nlon, ilon[b], si[b])
                    pltpu.async_copy(nlat, ilat[b], si[b])

        for b in (0, 1):
            pltpu.make_async_copy(obuf.at[b], out_dst(_NCHUNK - 2 + b),
                                  so[b]).wait()

    return k(tlon_flat, tlat_flat, idx_lon, idx_lat)


def _pack_table(table):
    tt = jnp.transpose(table.astype(jnp.bfloat16))
    lo = jax.lax.bitcast_convert_type(tt[0::2], jnp.uint16).astype(jnp.uint32)
    hi = jax.lax.bitcast_convert_type(tt[1::2], jnp.uint16).astype(jnp.uint32)
    packed = lo | (hi << 16)
    return jax.lax.bitcast_convert_type(packed, jnp.int32).reshape(-1)


def kernel(d_mat_lon, d_mat_lat, table_lon, table_lat):
    idx_lon, idx_lat = _compute_indices(d_mat_lon, d_mat_lat)
    tlon_flat = _pack_table(table_lon)
    tlat_flat = _pack_table(table_lat)
    out_acb = _sc_gather_sum(tlon_flat, tlat_flat,
                             idx_lon.reshape(-1), idx_lat.reshape(-1))
    return jnp.transpose(out_acb, (0, 2, 1))


# speedup vs baseline: 1.3018x; 1.3018x over previous
import functools
import math

import jax
import jax.numpy as jnp
from jax import lax
from jax.experimental import pallas as pl
from jax.experimental.pallas import tpu as pltpu
from jax.experimental.pallas import tpu_sc as plsc

RADIUS_EARTH = 6371.0
MIN_DIST = 1.0
MAX_DIST = 1000.0
N_DIST = 1024
EMB_DIM = 16
LOG_MIN = math.log(MIN_DIST / RADIUS_EARTH)
LOG_MAX = math.log(MAX_DIST / RADIUS_EARTH)

_N = 2048
_B = _N * _N
_ROW_BLK = 128
_V = N_DIST + 1
_NPAIR = EMB_DIM // 2
_TPACK = _NPAIR * _V

_NC = 2
_NS = 16
_NW = _NC * _NS
_BPW = _B // _NW
_CHUNK = 2048
_GRP = _CHUNK // 16
_NCHUNK = _BPW // _CHUNK
_PAD = EMB_DIM + 1


def _bucket_idx(d):
    sgn = jnp.sign(d)
    t = jnp.log(jnp.abs(d))
    t = jnp.clip(t, LOG_MIN, LOG_MAX)
    t = (t - LOG_MIN) / (LOG_MAX - LOG_MIN)
    t = t * (N_DIST / 2.0)
    half = N_DIST / 2.0
    t = jnp.where(sgn < 0, t + (half - 1.0), half - 1.0 - t)
    idx = t.astype(jnp.int32)
    idx = jnp.where(idx < 0, idx + (N_DIST + 1), idx)
    return idx


def _idx_body(dlon_ref, dlat_ref, ilon_ref, ilat_ref):
    ilon_ref[...] = _bucket_idx(dlon_ref[...]).reshape(-1)
    ilat_ref[...] = _bucket_idx(dlat_ref[...]).reshape(-1)


def _compute_indices(d_mat_lon, d_mat_lat):
    spec = pl.BlockSpec((_ROW_BLK, _N), lambda i: (i, 0))
    ospec = pl.BlockSpec((_ROW_BLK * _N,), lambda i: (i,))
    return pl.pallas_call(
        _idx_body,
        grid=(_N // _ROW_BLK,),
        in_specs=[spec, spec],
        out_specs=[ospec, ospec],
        out_shape=[
            jax.ShapeDtypeStruct((_B,), jnp.int32),
            jax.ShapeDtypeStruct((_B,), jnp.int32),
        ],
    )(d_mat_lon, d_mat_lat)


def _sc_gather_sum(tlon_flat, tlat_flat, idx_lon, idx_lat):
    mesh = plsc.VectorSubcoreMesh(core_axis_name="c", subcore_axis_name="s")

    @functools.partial(
        pl.kernel,
        out_type=jax.ShapeDtypeStruct((_N, EMB_DIM, _N), jnp.float32),
        mesh=mesh,
        compiler_params=pltpu.CompilerParams(needs_layout_passes=False),
        scratch_types=[
            pltpu.VMEM((_TPACK,), jnp.int32),
            pltpu.VMEM((_TPACK,), jnp.int32),
            pltpu.VMEM((_CHUNK,), jnp.int32),
            pltpu.VMEM((_CHUNK,), jnp.int32),
            pltpu.VMEM((_CHUNK,), jnp.int32),
            pltpu.VMEM((_CHUNK,), jnp.int32),
            pltpu.VMEM((2, EMB_DIM, _CHUNK), jnp.float32),
            pltpu.SemaphoreType.DMA,
            pltpu.SemaphoreType.DMA,
            pltpu.SemaphoreType.DMA,
            pltpu.SemaphoreType.DMA,
        ],
    )
    def k(tlon_hbm, tlat_hbm, ilon_hbm, ilat_hbm, out_hbm,
          tlon_v, tlat_v, ilon0, ilon1, ilat0, ilat1, obuf,
          si0, si1, so0, so1):
        si = (si0, si1)
        so = (so0, so1)
        ilon = (ilon0, ilon1)
        ilat = (ilat0, ilat1)
        wid = lax.axis_index("s") * _NC + lax.axis_index("c")
        base = wid * _BPW

        pltpu.sync_copy(tlon_hbm, tlon_v)
        pltpu.sync_copy(tlat_hbm, tlat_v)

        def idx_src(c):
            off = pl.multiple_of(base + c * _CHUNK, _CHUNK)
            return (ilon_hbm.at[pl.ds(off, _CHUNK)],
                    ilat_hbm.at[pl.ds(off, _CHUNK)])

        def out_dst(c):
            return out_hbm.at[wid * _NCHUNK + c]

        for b in (0, 1):
            slon, slat = idx_src(b)
            pltpu.async_copy(slon, ilon[b], si[b])
            pltpu.async_copy(slat, ilat[b], si[b])

        @pl.loop(0, _NCHUNK, step=2)
        def _pair(c0):
            for b in (0, 1):
                c = c0 + b
                slon, slat = idx_src(c)
                pltpu.make_async_copy(slon, ilon[b], si[b]).wait()
                pltpu.make_async_copy(slat, ilat[b], si[b]).wait()

                @pl.when(c >= 2)
                def _drain():
                    pltpu.make_async_copy(obuf.at[b], out_dst(c - 2),
                                          so[b]).wait()

                @plsc.parallel_loop(0, _GRP, 1, unroll=4)
                def _grp(g):
                    ivl = ilon[b][pl.ds(g * 16, 16)]
                    ivt = ilat[b][pl.ds(g * 16, 16)]
                    for d2 in range(_NPAIR):
                        gl = plsc.load_gather(tlon_v, [ivl + d2 * _V])
                        gt = plsc.load_gather(tlat_v, [ivt + d2 * _V])
                        s = (plsc.bitcast(gl, jnp.bfloat16)
                             + plsc.bitcast(gt, jnp.bfloat16))
                        s0, s1 = plsc.unpack(
                            s, format=plsc.PackFormat.INTERLEAVED,
                            preferred_element_type=jnp.float32)
                        obuf[b, 2 * d2, pl.ds(g * 16, 16)] = s0
                        obuf[b, 2 * d2 + 1, pl.ds(g * 16, 16)] = s1

                pltpu.async_copy(obuf.at[b], out_dst(c), so[b])

                @pl.when(c + 2 < _NCHUNK)
                def _prefetch():
                    nlon, nlat = idx_src(c + 2)
                    pltpu.async_copy(nlon, ilon[b], si[b])
                    pltpu.async_copy(nlat, ilat[b], si[b])

        for b in (0, 1):
            pltpu.make_async_copy(obuf.at[b], out_dst(_NCHUNK - 2 + b),
                                  so[b]).wait()

    return k(tlon_flat, tlat_flat, idx_lon, idx_lat)


def _pack_table(table):
    tt = jnp.transpose(table.astype(jnp.bfloat16))
    lo = jax.lax.bitcast_convert_type(tt[0::2], jnp.uint16).astype(jnp.uint32)
    hi = jax.lax.bitcast_convert_type(tt[1::2], jnp.uint16).astype(jnp.uint32)
    packed = lo | (hi << 16)
    return jax.lax.bitcast_convert_type(packed, jnp.int32).reshape(-1)


def kernel(d_mat_lon, d_mat_lat, table_lon, table_lat):
    idx_lon, idx_lat = _compute_indices(d_mat_lon, d_mat_lat)
    tlon_flat = _pack_table(table_lon)
    tlat_flat = _pack_table(table_lat)
    out_acb = _sc_gather_sum(tlon_flat, tlat_flat, idx_lon, idx_lat)
    return jnp.transpose(out_acb, (0, 2, 1))
